# baseline (speedup 1.0000x reference)
import functools
import math

import jax
import jax.numpy as jnp
from jax import lax
from jax.experimental import pallas as pl
from jax.experimental.pallas import tpu as pltpu, tpu_sc as plsc

N0 = 4096
E = 65536
H = 128
K0 = 2048
K1 = 1024

_SC_MESH = plsc.VectorSubcoreMesh(core_axis_name="c", subcore_axis_name="s")
_NSC = 2
_NT = 16
_CH = 128
_EPT = E // (_NSC * _NT)


def _dinv_of(deg):
    return jnp.where(deg > 0, lax.rsqrt(jnp.maximum(deg, 1e-12)), 0.0)


def _zero16(ref, n, row=None):
    def zf(i, _):
        if row is None:
            ref[pl.ds(i * 16, 16)] = jnp.zeros((16,), jnp.float32)
        else:
            ref[row, pl.ds(i * 16, 16)] = jnp.zeros((16,), jnp.float32)
        return 0
    lax.fori_loop(0, n // 16, zf, 0)


@functools.partial(
    pl.kernel, mesh=_SC_MESH,
    out_type=jax.ShapeDtypeStruct((_NSC, N0), jnp.float32),
    scratch_types=[
        pltpu.VMEM((_CH,), jnp.int32),
        pltpu.VMEM((_CH,), jnp.float32),
        pltpu.VMEM((N0 // _NT,), jnp.float32),
        pltpu.VMEM_SHARED((N0,), jnp.float32),
    ],
)
def _sc_deg(dst_hbm, w_hbm, out_hbm, idx_v, w_v, zd_v, deg_sh):
    cid = lax.axis_index("c")
    sid = lax.axis_index("s")
    sl = N0 // _NT
    _zero16(zd_v, sl)
    pltpu.sync_copy(zd_v, deg_sh.at[pl.ds(sid * sl, sl)])
    plsc.subcore_barrier()

    base0 = cid * (E // _NSC) + sid * _EPT

    def chunk(ci, _):
        base = base0 + ci * _CH
        pltpu.sync_copy(dst_hbm.at[pl.ds(base, _CH)], idx_v)
        pltpu.sync_copy(w_hbm.at[pl.ds(base, _CH)], w_v)
        pltpu.sync_copy(w_v, deg_sh.at[idx_v], add=True)
        return 0
    lax.fori_loop(0, _EPT // _CH, chunk, 0)
    plsc.subcore_barrier()
    pltpu.sync_copy(deg_sh.at[pl.ds(sid * sl, sl)],
                    out_hbm.at[cid, pl.ds(sid * sl, sl)])


@functools.partial(
    pl.kernel, mesh=_SC_MESH,
    out_type=jax.ShapeDtypeStruct((_NSC, N0, H), jnp.float32),
    scratch_types=[
        pltpu.VMEM((_CH,), jnp.int32),
        pltpu.VMEM((_CH,), jnp.int32),
        pltpu.VMEM((_CH,), jnp.float32),
        pltpu.VMEM((_CH, H), jnp.float32),
        pltpu.VMEM((16, H), jnp.float32),
        pltpu.VMEM_SHARED((N0, H), jnp.float32),
        pltpu.SemaphoreType.DMA,
    ],
)
def _sc_spmm(z_hbm, src_hbm, dst_hbm, w_hbm, out_hbm,
             src_v, dst_v, w_v, rows_v, zb_v, agg_sh, sem):
    cid = lax.axis_index("c")
    sid = lax.axis_index("s")
    sl = N0 // _NT
    def zf(i, _):
        for f in range(H // 16):
            zb_v[i, pl.ds(f * 16, 16)] = jnp.zeros((16,), jnp.float32)
        return 0
    lax.fori_loop(0, 16, zf, 0)

    def zs(i, _):
        pltpu.sync_copy(zb_v, agg_sh.at[pl.ds(sid * sl + i * 16, 16)])
        return 0
    lax.fori_loop(0, sl // 16, zs, 0)
    plsc.subcore_barrier()

    base0 = cid * (E // _NSC) + sid * _EPT

    def chunk(ci, _):
        base = base0 + ci * _CH
        pltpu.sync_copy(src_hbm.at[pl.ds(base, _CH)], src_v)
        pltpu.sync_copy(dst_hbm.at[pl.ds(base, _CH)], dst_v)
        pltpu.sync_copy(w_hbm.at[pl.ds(base, _CH)], w_v)
        pltpu.async_copy(z_hbm.at[src_v], rows_v, sem).wait()

        def scale(g, _):
            wg = w_v[pl.ds(g * 16, 16)]
            for l in range(16):
                e = g * 16 + l
                wb = wg[l]
                for f in range(H // 16):
                    rows_v[e, pl.ds(f * 16, 16)] = (
                        rows_v[e, pl.ds(f * 16, 16)] * wb)
            return 0
        lax.fori_loop(0, _CH // 16, scale, 0)
        pltpu.sync_copy(rows_v, agg_sh.at[dst_v], add=True)
        return 0
    lax.fori_loop(0, _EPT // _CH, chunk, 0)
    plsc.subcore_barrier()
    pltpu.sync_copy(agg_sh.at[pl.ds(sid * sl, sl)],
                    out_hbm.at[cid, pl.ds(sid * sl, sl)])


_SLAB = 256
_NSLAB = K0 // _SLAB
_ECH = E // _CH
_CPT = _ECH // _NT


@functools.partial(
    pl.kernel, mesh=_SC_MESH,
    out_type=[
        jax.ShapeDtypeStruct((K0 * N0,), jnp.float32),
        jax.ShapeDtypeStruct((K0 * N0,), jnp.float32),
    ],
    scratch_types=[
        pltpu.VMEM((_CPT, _CH), jnp.int32),
        pltpu.VMEM((_CPT, _CH), jnp.int32),
        pltpu.VMEM((_CPT, _CH), jnp.float32),
        pltpu.VMEM((_CPT, _CH), jnp.int32),
        pltpu.VMEM((_CPT, _CH), jnp.int32),
        pltpu.VMEM((_CPT, _CH), jnp.int32),
        pltpu.VMEM((4096,), jnp.float32),
        pltpu.VMEM((1, 16), jnp.int32),
        pltpu.VMEM((1, 16), jnp.float32),
        pltpu.VMEM_SHARED(((_SLAB + 1) * N0,), jnp.float32),
        pltpu.SemaphoreType.DMA,
    ],
)
def _sc_factors(src2, dst2, w2, rkd2, rks2, perm_hbm, r_out, ct_out,
                src_t, dst_t, w_t, rkd_t, rks_t, idx_t, zb_v, sidx_v,
                ones_v, sh, sem):
    cid = lax.axis_index("c")
    sid = lax.axis_index("s")
    crow = sid * _CPT

    pltpu.sync_copy(src2.at[pl.ds(crow, _CPT)], src_t)
    pltpu.sync_copy(dst2.at[pl.ds(crow, _CPT)], dst_t)
    pltpu.sync_copy(w2.at[pl.ds(crow, _CPT)], w_t)
    pltpu.sync_copy(rkd2.at[pl.ds(crow, _CPT)], rkd_t)
    pltpu.sync_copy(rks2.at[pl.ds(crow, _CPT)], rks_t)
    _zero16(zb_v, 4096)
    ones_v[0, :] = jnp.ones((16,), jnp.float32)
    dump = _SLAB * N0

    for mat in range(2):
        out = r_out if mat == 0 else ct_out

        def do_round(r, _):
            slab = r * _NSC + cid
            rbase = slab * _SLAB
            def zs(i, _):
                pltpu.sync_copy(
                    zb_v, sh.at[pl.ds((sid * 16 + i) * N0, N0)])
                return 0
            lax.fori_loop(0, 16, zs, 0)
            plsc.subcore_barrier()

            def chunk(c, _):
                def grp(g, _):
                    rk = (rkd_t if mat == 0 else rks_t)[c, pl.ds(g * 16, 16)]
                    col = (src_t if mat == 0 else dst_t)[c, pl.ds(g * 16, 16)]
                    other = (dst_t if mat == 0 else src_t)[c, pl.ds(g * 16, 16)]
                    rloc = rk - rbase
                    ok = ((rk >= 0) & (col != other)
                          & (rloc >= 0) & (rloc < _SLAB))
                    idx_t[c, pl.ds(g * 16, 16)] = jnp.where(
                        ok, rloc * N0 + col, dump)
                    return 0
                lax.fori_loop(0, _CH // 16, grp, 0)
                pltpu.sync_copy(w_t.at[c], sh.at[idx_t.at[c]], add=True)
                return 0
            lax.fori_loop(0, _CPT, chunk, 0)

            pltpu.sync_copy(perm_hbm.at[pl.ds(rbase + sid * 16, 16)],
                            sidx_v.at[0])
            pv = sidx_v[0, :]
            rr = lax.iota(jnp.int32, 16) + sid * 16
            sidx_v[0, :] = rr * N0 + pv
            pltpu.sync_copy(ones_v.at[0], sh.at[sidx_v.at[0]], add=True)
            plsc.subcore_barrier()

            pltpu.sync_copy(
                sh.at[pl.ds(sid * 16 * N0, 16 * N0)],
                out.at[pl.ds((rbase + sid * 16) * N0, 16 * N0)])
            plsc.subcore_barrier()
            return 0
        lax.fori_loop(0, _NSLAB // _NSC, do_round, 0)


def _make_gather(k, width, naux):
    rpt = k // (_NSC * _NT)
    nch = max(1, rpt // _CH)
    cpr = min(rpt, _CH)

    @functools.partial(
        pl.kernel, mesh=_SC_MESH,
        out_type=([jax.ShapeDtypeStruct((k, width), jnp.float32)]
                  + [jax.ShapeDtypeStruct((k,), jnp.float32)] * naux),
        scratch_types=[
            pltpu.VMEM((cpr,), jnp.int32),
            pltpu.VMEM((cpr, width), jnp.float32),
            pltpu.VMEM((cpr,), jnp.float32),
            pltpu.SemaphoreType.DMA,
        ],
    )
    def gk(idx_hbm, tab_hbm, *rest):
        aux = rest[:naux]
        out_hbm = rest[naux]
        aux_out = rest[naux + 1:naux + 1 + naux]
        idx_v, rows_v, el_v, sem = rest[naux + 1 + naux:]
        cid = lax.axis_index("c")
        sid = lax.axis_index("s")
        base0 = (cid * _NT + sid) * rpt

        def chunk(ci, _):
            base = base0 + ci * cpr
            pltpu.sync_copy(idx_hbm.at[pl.ds(base, cpr)], idx_v)
            pltpu.async_copy(tab_hbm.at[idx_v], rows_v, sem).wait()
            pltpu.sync_copy(rows_v, out_hbm.at[pl.ds(base, cpr)])
            for j in range(naux):
                pltpu.async_copy(aux[j].at[idx_v], el_v, sem).wait()
                pltpu.sync_copy(el_v, aux_out[j].at[pl.ds(base, cpr)])
            return 0
        lax.fori_loop(0, nch, chunk, 0)

    return gk


_gather_x0 = _make_gather(K0, H, 2)
_gather_x1 = _make_gather(K1, H, 1)
_gather_a1 = _make_gather(K1, K0, 0)


def _make_unpool(m, k):
    half = m // 2
    sl = half // _NT
    rpt = k // _NT
    nch = max(1, rpt // _CH)
    cpr = min(rpt, _CH)

    @functools.partial(
        pl.kernel, mesh=_SC_MESH,
        out_type=jax.ShapeDtypeStruct((m, H), jnp.float32),
        scratch_types=[
            pltpu.VMEM((cpr,), jnp.int32),
            pltpu.VMEM((cpr, H), jnp.float32),
            pltpu.VMEM_SHARED((half + 1, H), jnp.float32),
            pltpu.SemaphoreType.DMA,
        ],
    )
    def uk(base_hbm, u_hbm, perm_hbm, out_hbm, idx_v, rows_v, sh, sem):
        cid = lax.axis_index("c")
        sid = lax.axis_index("s")
        hbase = cid * half
        pltpu.sync_copy(base_hbm.at[pl.ds(hbase + sid * sl, sl)],
                        sh.at[pl.ds(sid * sl, sl)])
        plsc.subcore_barrier()

        def chunk(ci, _):
            base = sid * rpt + ci * cpr
            pltpu.sync_copy(perm_hbm.at[pl.ds(base, cpr)], idx_v)
            pltpu.sync_copy(u_hbm.at[pl.ds(base, cpr)], rows_v)
            def grp(g, _):
                pv = idx_v[pl.ds(g * 16, 16)]
                loc = pv - hbase
                ok = (loc >= 0) & (loc < half)
                idx_v[pl.ds(g * 16, 16)] = jnp.where(ok, loc, half)
                return 0
            lax.fori_loop(0, cpr // 16, grp, 0)
            pltpu.sync_copy(rows_v, sh.at[idx_v], add=True)
            return 0
        lax.fori_loop(0, nch, chunk, 0)
        plsc.subcore_barrier()
        pltpu.sync_copy(sh.at[pl.ds(sid * sl, sl)],
                        out_hbm.at[pl.ds(hbase + sid * sl, sl)])

    return uk


_unpool1 = _make_unpool(K0, K1)
_unpool0 = _make_unpool(N0, K0)


def _fin_body(p_ref, deg_ref, b_ref, pv_ref, y_ref, s_ref, th_ref):
    deg = deg_ref[0, :] + deg_ref[1, :]
    dinv = _dinv_of(deg)
    acc = p_ref[0] + p_ref[1]
    y = jnp.maximum(acc * dinv[:, None] + b_ref[...], 0.0)
    y_ref[...] = y
    pv = pv_ref[...]
    pn = pv / jnp.sqrt(jnp.sum(pv * pv))
    s = jnp.dot(y, pn.reshape(H, 1), preferred_element_type=jnp.float32)
    s_ref[...] = s
    th_ref[...] = jnp.tanh(s)


def _fin(P, degP, b, pvec, bm=1024):
    m = P.shape[1]
    return pl.pallas_call(
        _fin_body,
        grid=(m // bm,),
        in_specs=[
            pl.BlockSpec((2, bm, H), lambda i: (0, i, 0)),
            pl.BlockSpec((2, bm), lambda i: (0, i)),
            pl.BlockSpec((1, H), lambda i: (0, 0)),
            pl.BlockSpec((1, H), lambda i: (0, 0)),
        ],
        out_specs=[
            pl.BlockSpec((bm, H), lambda i: (i, 0)),
            pl.BlockSpec((bm, 1), lambda i: (i, 0)),
            pl.BlockSpec((bm, 1), lambda i: (i, 0)),
        ],
        out_shape=[
            jax.ShapeDtypeStruct((m, H), jnp.float32),
            jax.ShapeDtypeStruct((m, 1), jnp.float32),
            jax.ShapeDtypeStruct((m, 1), jnp.float32),
        ],
    )(P, degP, b.reshape(1, H), pvec.reshape(1, H))


def _sq_body(nsteps, inject, twout, r_ref, ct_ref, pr_ref, pc_ref,
             a_ref, deg_ref, *maybe_at):
    i, j, k = pl.program_id(0), pl.program_id(1), pl.program_id(2)

    @pl.when(k == 0)
    def _():
        a_ref[...] = jnp.zeros_like(a_ref)

    rb = r_ref[...]
    cb = ct_ref[...]
    if inject:
        bk = rb.shape[1]
        colk = lax.broadcasted_iota(jnp.int32, (rb.shape[0], bk), 1) + k * bk
        rb = rb + jnp.where(colk == pr_ref[...][:, None], 1.0, 0.0)
        colk2 = lax.broadcasted_iota(jnp.int32, (cb.shape[0], bk), 1) + k * bk
        cb = cb + jnp.where(colk2 == pc_ref[...][:, None], 1.0, 0.0)
    a_ref[...] += lax.dot_general(
        rb, cb, (((1,), (1,)), ((), ())), preferred_element_type=jnp.float32)

    @pl.when(k == nsteps - 1)
    def _():
        blk = a_ref[...]
        bm, bn = blk.shape
        rows = lax.broadcasted_iota(jnp.int32, (bm, bn), 0) + i * bm
        cols = lax.broadcasted_iota(jnp.int32, (bm, bn), 1) + j * bn
        blk = jnp.where(rows == cols, 0.0, blk)
        a_ref[...] = blk
        if twout:
            maybe_at[0][...] = blk.T
        rs = jnp.sum(blk, axis=1)

        @pl.when(j == 0)
        def _():
            deg_ref[...] = rs

        @pl.when(j != 0)
        def _():
            deg_ref[...] += rs


def _square_pool(Rm, CTm, permv=None, twout=False, bm=512, bn=512, bk=2048):
    m, K = Rm.shape
    nsteps = K // bk
    grid = (m // bm, m // bn, nsteps)
    inject = permv is not None
    if permv is None:
        permv = jnp.zeros((m,), jnp.int32)
    out_specs = [
        pl.BlockSpec((bm, bn), lambda i, j, k: (i, j)),
        pl.BlockSpec((bm,), lambda i, j, k: (i,)),
    ]
    out_shape = [
        jax.ShapeDtypeStruct((m, m), jnp.float32),
        jax.ShapeDtypeStruct((m,), jnp.float32),
    ]
    if twout:
        out_specs.append(pl.BlockSpec((bn, bm), lambda i, j, k: (j, i)))
        out_shape.append(jax.ShapeDtypeStruct((m, m), jnp.float32))
    return pl.pallas_call(
        functools.partial(_sq_body, nsteps, inject, twout),
        grid=grid,
        in_specs=[
            pl.BlockSpec((bm, bk), lambda i, j, k: (i, k)),
            pl.BlockSpec((bn, bk), lambda i, j, k: (j, k)),
            pl.BlockSpec((bm,), lambda i, j, k: (i,)),
            pl.BlockSpec((bn,), lambda i, j, k: (j,)),
        ],
        out_specs=out_specs,
        out_shape=out_shape,
    )(Rm, CTm, permv, permv)


def _conv_body(a_ref, z_ref, degk_ref, degi_ref, b_ref, pv_ref,
               y_ref, s_ref, th_ref):
    dinvk = _dinv_of(degk_ref[...])
    zs = z_ref[...] * dinvk[:, None]
    acc = jnp.dot(a_ref[...], zs, preferred_element_type=jnp.float32)
    dinvi = _dinv_of(degi_ref[...])
    y = jnp.maximum(acc * dinvi[:, None] + b_ref[...], 0.0)
    y_ref[...] = y
    pv = pv_ref[...]
    pn = pv / jnp.sqrt(jnp.sum(pv * pv))
    s = jnp.dot(y, pn.reshape(H, 1), preferred_element_type=jnp.float32)
    s_ref[...] = s
    th_ref[...] = jnp.tanh(s)


def _conv_dense(A, Z, deg, b, pvec, bm=512):
    m = A.shape[0]
    return pl.pallas_call(
        _conv_body,
        grid=(m // bm,),
        in_specs=[
            pl.BlockSpec((bm, m), lambda i: (i, 0)),
            pl.BlockSpec((m, H), lambda i: (0, 0)),
            pl.BlockSpec((m,), lambda i: (0,)),
            pl.BlockSpec((bm,), lambda i: (i,)),
            pl.BlockSpec((1, H), lambda i: (0, 0)),
            pl.BlockSpec((1, H), lambda i: (0, 0)),
        ],
        out_specs=[
            pl.BlockSpec((bm, H), lambda i: (i, 0)),
            pl.BlockSpec((bm, 1), lambda i: (i, 0)),
            pl.BlockSpec((bm, 1), lambda i: (i, 0)),
        ],
        out_shape=[
            jax.ShapeDtypeStruct((m, H), jnp.float32),
            jax.ShapeDtypeStruct((m, 1), jnp.float32),
            jax.ShapeDtypeStruct((m, 1), jnp.float32),
        ],
    )(A, Z, deg, deg, b.reshape(1, H), pvec.reshape(1, H))


def _mm_body(x_ref, w_ref, si_ref, so_ref, y_ref):
    y = jnp.dot(x_ref[...] * si_ref[...][:, None], w_ref[...],
                preferred_element_type=jnp.float32)
    y_ref[...] = y * so_ref[...][:, None]


def _mm(X, W, s_in=None, s_out=None, bm=1024):
    m, f = X.shape
    if s_in is None:
        s_in = jnp.ones((m,), jnp.float32)
    if s_out is None:
        s_out = jnp.ones((m,), jnp.float32)
    return pl.pallas_call(
        _mm_body,
        grid=(m // bm,),
        in_specs=[
            pl.BlockSpec((bm, f), lambda i: (i, 0)),
            pl.BlockSpec((f, W.shape[1]), lambda i: (0, 0)),
            pl.BlockSpec((bm,), lambda i: (i,)),
            pl.BlockSpec((bm,), lambda i: (i,)),
        ],
        out_specs=pl.BlockSpec((bm, W.shape[1]), lambda i: (i, 0)),
        out_shape=jax.ShapeDtypeStruct((m, W.shape[1]), jnp.float32),
    )(X, W, s_in, s_out)


def _topk_set(score, k):
    _, perm = lax.top_k(score, k)
    return jnp.sort(perm)


def kernel(x, edge_index, edge_weight, W0, b0, W1, b1, W2, b2, U0, c0, U1, c1, p0, p1):
    xf = x.reshape(N0, H)
    dst, src = edge_index[1], edge_index[0]

    degP = _sc_deg(dst, edge_weight)
    deg0 = degP[0] + degP[1]
    dinv0 = _dinv_of(deg0)

    z0p = _mm(xf, W0, s_out=dinv0)
    P = _sc_spmm(z0p, src, dst, edge_weight)
    x0, s0, th0 = _fin(P, degP, b0, p0)
    score0 = s0.reshape(N0)

    perm0 = _topk_set(score0, K0)
    rank0 = jnp.full((N0,), -1, jnp.int32).at[perm0].set(
        jnp.arange(K0, dtype=jnp.int32))

    src2 = src.reshape(_ECH, _CH)
    dst2 = dst.reshape(_ECH, _CH)
    w2 = edge_weight.reshape(_ECH, _CH)
    rkd2 = rank0[dst].reshape(_ECH, _CH)
    rks2 = rank0[src].reshape(_ECH, _CH)
    R0f, CT0f = _sc_factors(src2, dst2, w2, rkd2, rks2, perm0)
    R0 = R0f.reshape(K0, N0)
    CT0 = CT0f.reshape(K0, N0)

    A1, deg1, A1T = _square_pool(R0, CT0, twout=True)

    x0g, t0, di0g = _gather_x0(perm0, x0, th0.reshape(N0), dinv0)
    x1, s1, th1 = _conv_dense(A1, _mm(x0g, W1, s_in=t0), deg1, b1, p1)
    score1 = s1.reshape(K0)

    perm1 = _topk_set(score1, K1)

    x1g, t1 = _gather_x1(perm1, x1, th1.reshape(K0))

    R1 = _gather_a1(perm1, A1)[0]
    CT1 = _gather_a1(perm1, A1T)[0]

    A2, deg2 = _square_pool(R1, CT1, permv=perm1, bm=512, bn=512, bk=2048)

    x2, _, _ = _conv_dense(A2, _mm(x1g, W2, s_in=t1, bm=512), deg2, b2, p1)

    z1 = _unpool1(_mm(x1, U0[:H]), _mm(x2, U0[H:]), perm1)
    x3, _, _ = _conv_dense(A1, z1, deg1, c0, p1)

    u0 = _mm(x3, U1[H:], s_out=di0g)
    zup = _unpool0(_mm(x0, U1[:H], s_out=dinv0), u0, perm0)
    Q = _sc_spmm(zup, src, dst, edge_weight)
    out, _, _ = _fin(Q, degP, c1, p1)
    return out.reshape(1, N0, H)


# speedup vs baseline: 1.0217x; 1.0217x over previous
import functools
import math

import jax
import jax.numpy as jnp
from jax import lax
from jax.experimental import pallas as pl
from jax.experimental.pallas import tpu as pltpu, tpu_sc as plsc

N0 = 4096
E = 65536
H = 128
K0 = 2048
K1 = 1024

_SC_MESH = plsc.VectorSubcoreMesh(core_axis_name="c", subcore_axis_name="s")
_NSC = 2
_NT = 16
_CH = 128
_EPT = E // (_NSC * _NT)


def _dinv_of(deg):
    return jnp.where(deg > 0, lax.rsqrt(jnp.maximum(deg, 1e-12)), 0.0)


def _zero16(ref, n, row=None):
    def zf(i, _):
        if row is None:
            ref[pl.ds(i * 16, 16)] = jnp.zeros((16,), jnp.float32)
        else:
            ref[row, pl.ds(i * 16, 16)] = jnp.zeros((16,), jnp.float32)
        return 0
    lax.fori_loop(0, n // 16, zf, 0)


@functools.partial(
    pl.kernel, mesh=_SC_MESH,
    out_type=jax.ShapeDtypeStruct((_NSC, N0), jnp.float32),
    scratch_types=[
        pltpu.VMEM((_CH,), jnp.int32),
        pltpu.VMEM((_CH,), jnp.float32),
        pltpu.VMEM((N0 // _NT,), jnp.float32),
        pltpu.VMEM_SHARED((N0,), jnp.float32),
    ],
)
def _sc_deg(dst_hbm, w_hbm, out_hbm, idx_v, w_v, zd_v, deg_sh):
    cid = lax.axis_index("c")
    sid = lax.axis_index("s")
    sl = N0 // _NT
    _zero16(zd_v, sl)
    pltpu.sync_copy(zd_v, deg_sh.at[pl.ds(sid * sl, sl)])
    plsc.subcore_barrier()

    base0 = cid * (E // _NSC) + sid * _EPT

    def chunk(ci, _):
        base = base0 + ci * _CH
        pltpu.sync_copy(dst_hbm.at[pl.ds(base, _CH)], idx_v)
        pltpu.sync_copy(w_hbm.at[pl.ds(base, _CH)], w_v)
        pltpu.sync_copy(w_v, deg_sh.at[idx_v], add=True)
        return 0
    lax.fori_loop(0, _EPT // _CH, chunk, 0)
    plsc.subcore_barrier()
    pltpu.sync_copy(deg_sh.at[pl.ds(sid * sl, sl)],
                    out_hbm.at[cid, pl.ds(sid * sl, sl)])


@functools.partial(
    pl.kernel, mesh=_SC_MESH,
    out_type=jax.ShapeDtypeStruct((_NSC, N0, H), jnp.float32),
    scratch_types=[
        pltpu.VMEM((_EPT,), jnp.int32),
        pltpu.VMEM((_EPT,), jnp.int32),
        pltpu.VMEM((_EPT,), jnp.float32),
        pltpu.VMEM((2, _CH, H), jnp.float32),
        pltpu.VMEM((16, H), jnp.float32),
        pltpu.VMEM_SHARED((N0, H), jnp.float32),
        pltpu.SemaphoreType.DMA,
        pltpu.SemaphoreType.DMA,
    ],
)
def _sc_spmm(z_hbm, src_hbm, dst_hbm, w_hbm, out_hbm,
             src_v, dst_v, w_v, rows_v, zb_v, agg_sh, sem0, sem1):
    cid = lax.axis_index("c")
    sid = lax.axis_index("s")
    sl = N0 // _NT
    def zf(i, _):
        for f in range(H // 16):
            zb_v[i, pl.ds(f * 16, 16)] = jnp.zeros((16,), jnp.float32)
        return 0
    lax.fori_loop(0, 16, zf, 0)

    def zs(i, _):
        pltpu.sync_copy(zb_v, agg_sh.at[pl.ds(sid * sl + i * 16, 16)])
        return 0
    lax.fori_loop(0, sl // 16, zs, 0)
    plsc.subcore_barrier()

    base0 = cid * (E // _NSC) + sid * _EPT
    pltpu.sync_copy(src_hbm.at[pl.ds(base0, _EPT)], src_v)
    pltpu.sync_copy(dst_hbm.at[pl.ds(base0, _EPT)], dst_v)
    pltpu.sync_copy(w_hbm.at[pl.ds(base0, _EPT)], w_v)

    nch = _EPT // _CH
    sems = (sem0, sem1)

    def issue(ci, slot):
        return pltpu.async_copy(
            z_hbm.at[src_v.at[pl.ds(ci * _CH, _CH)]],
            rows_v.at[slot], sems[slot])

    h = issue(0, 0)
    for ci in range(nch):
        slot = ci % 2
        hn = issue(ci + 1, 1 - slot) if ci + 1 < nch else None
        h.wait()

        def scale(g, _):
            wg = w_v[pl.ds(ci * _CH + g * 16, 16)]
            for l in range(16):
                e = g * 16 + l
                wb = wg[l]
                for f in range(H // 16):
                    rows_v[slot, e, pl.ds(f * 16, 16)] = (
                        rows_v[slot, e, pl.ds(f * 16, 16)] * wb)
            return 0
        lax.fori_loop(0, _CH // 16, scale, 0)
        pltpu.sync_copy(rows_v.at[slot],
                        agg_sh.at[dst_v.at[pl.ds(ci * _CH, _CH)]], add=True)
        h = hn
    plsc.subcore_barrier()
    pltpu.sync_copy(agg_sh.at[pl.ds(sid * sl, sl)],
                    out_hbm.at[cid, pl.ds(sid * sl, sl)])


_SLAB = 256
_NSLAB = K0 // _SLAB
_ECH = E // _CH
_CPT = _ECH // _NT


@functools.partial(
    pl.kernel, mesh=_SC_MESH,
    out_type=[
        jax.ShapeDtypeStruct((K0 * N0,), jnp.float32),
        jax.ShapeDtypeStruct((K0 * N0,), jnp.float32),
    ],
    scratch_types=[
        pltpu.VMEM((_CPT, _CH), jnp.int32),
        pltpu.VMEM((_CPT, _CH), jnp.int32),
        pltpu.VMEM((_CPT, _CH), jnp.float32),
        pltpu.VMEM((_CPT, _CH), jnp.int32),
        pltpu.VMEM((_CPT, _CH), jnp.int32),
        pltpu.VMEM((_CPT, _CH), jnp.int32),
        pltpu.VMEM((4096,), jnp.float32),
        pltpu.VMEM((1, 16), jnp.int32),
        pltpu.VMEM((1, 16), jnp.float32),
        pltpu.VMEM_SHARED(((_SLAB + 1) * N0,), jnp.float32),
        pltpu.SemaphoreType.DMA,
    ],
)
def _sc_factors(src2, dst2, w2, rkd2, rks2, perm_hbm, r_out, ct_out,
                src_t, dst_t, w_t, rkd_t, rks_t, idx_t, zb_v, sidx_v,
                ones_v, sh, sem):
    cid = lax.axis_index("c")
    sid = lax.axis_index("s")
    crow = sid * _CPT

    pltpu.sync_copy(src2.at[pl.ds(crow, _CPT)], src_t)
    pltpu.sync_copy(dst2.at[pl.ds(crow, _CPT)], dst_t)
    pltpu.sync_copy(w2.at[pl.ds(crow, _CPT)], w_t)
    pltpu.sync_copy(rkd2.at[pl.ds(crow, _CPT)], rkd_t)
    pltpu.sync_copy(rks2.at[pl.ds(crow, _CPT)], rks_t)
    _zero16(zb_v, 4096)
    ones_v[0, :] = jnp.ones((16,), jnp.float32)
    dump = _SLAB * N0

    for mat in range(2):
        out = r_out if mat == 0 else ct_out

        def do_round(r, _):
            slab = r * _NSC + cid
            rbase = slab * _SLAB
            def zs(i, _):
                pltpu.sync_copy(
                    zb_v, sh.at[pl.ds((sid * 16 + i) * N0, N0)])
                return 0
            lax.fori_loop(0, 16, zs, 0)
            plsc.subcore_barrier()

            def chunk(c, _):
                def grp(g, _):
                    rk = (rkd_t if mat == 0 else rks_t)[c, pl.ds(g * 16, 16)]
                    col = (src_t if mat == 0 else dst_t)[c, pl.ds(g * 16, 16)]
                    other = (dst_t if mat == 0 else src_t)[c, pl.ds(g * 16, 16)]
                    rloc = rk - rbase
                    ok = ((rk >= 0) & (col != other)
                          & (rloc >= 0) & (rloc < _SLAB))
                    idx_t[c, pl.ds(g * 16, 16)] = jnp.where(
                        ok, rloc * N0 + col, dump)
                    return 0
                lax.fori_loop(0, _CH // 16, grp, 0)
                pltpu.sync_copy(w_t.at[c], sh.at[idx_t.at[c]], add=True)
                return 0
            lax.fori_loop(0, _CPT, chunk, 0)

            pltpu.sync_copy(perm_hbm.at[pl.ds(rbase + sid * 16, 16)],
                            sidx_v.at[0])
            pv = sidx_v[0, :]
            rr = lax.iota(jnp.int32, 16) + sid * 16
            sidx_v[0, :] = rr * N0 + pv
            pltpu.sync_copy(ones_v.at[0], sh.at[sidx_v.at[0]], add=True)
            plsc.subcore_barrier()

            pltpu.sync_copy(
                sh.at[pl.ds(sid * 16 * N0, 16 * N0)],
                out.at[pl.ds((rbase + sid * 16) * N0, 16 * N0)])
            plsc.subcore_barrier()
            return 0
        lax.fori_loop(0, _NSLAB // _NSC, do_round, 0)


def _make_gather(k, width, naux):
    rpt = k // (_NSC * _NT)
    nch = max(1, rpt // _CH)
    cpr = min(rpt, _CH)

    @functools.partial(
        pl.kernel, mesh=_SC_MESH,
        out_type=([jax.ShapeDtypeStruct((k, width), jnp.float32)]
                  + [jax.ShapeDtypeStruct((k,), jnp.float32)] * naux),
        scratch_types=[
            pltpu.VMEM((cpr,), jnp.int32),
            pltpu.VMEM((cpr, width), jnp.float32),
            pltpu.VMEM((cpr,), jnp.float32),
            pltpu.SemaphoreType.DMA,
        ],
    )
    def gk(idx_hbm, tab_hbm, *rest):
        aux = rest[:naux]
        out_hbm = rest[naux]
        aux_out = rest[naux + 1:naux + 1 + naux]
        idx_v, rows_v, el_v, sem = rest[naux + 1 + naux:]
        cid = lax.axis_index("c")
        sid = lax.axis_index("s")
        base0 = (cid * _NT + sid) * rpt

        def chunk(ci, _):
            base = base0 + ci * cpr
            pltpu.sync_copy(idx_hbm.at[pl.ds(base, cpr)], idx_v)
            pltpu.async_copy(tab_hbm.at[idx_v], rows_v, sem).wait()
            pltpu.sync_copy(rows_v, out_hbm.at[pl.ds(base, cpr)])
            for j in range(naux):
                pltpu.async_copy(aux[j].at[idx_v], el_v, sem).wait()
                pltpu.sync_copy(el_v, aux_out[j].at[pl.ds(base, cpr)])
            return 0
        lax.fori_loop(0, nch, chunk, 0)

    return gk


_gather_x0 = _make_gather(K0, H, 2)
_gather_x1 = _make_gather(K1, H, 1)
_gather_a1 = _make_gather(K1, K0, 0)


def _make_unpool(m, k):
    half = m // 2
    sl = half // _NT
    rpt = k // _NT
    nch = max(1, rpt // _CH)
    cpr = min(rpt, _CH)

    @functools.partial(
        pl.kernel, mesh=_SC_MESH,
        out_type=jax.ShapeDtypeStruct((m, H), jnp.float32),
        scratch_types=[
            pltpu.VMEM((cpr,), jnp.int32),
            pltpu.VMEM((cpr, H), jnp.float32),
            pltpu.VMEM_SHARED((half + 1, H), jnp.float32),
            pltpu.SemaphoreType.DMA,
        ],
    )
    def uk(base_hbm, u_hbm, perm_hbm, out_hbm, idx_v, rows_v, sh, sem):
        cid = lax.axis_index("c")
        sid = lax.axis_index("s")
        hbase = cid * half
        pltpu.sync_copy(base_hbm.at[pl.ds(hbase + sid * sl, sl)],
                        sh.at[pl.ds(sid * sl, sl)])
        plsc.subcore_barrier()

        def chunk(ci, _):
            base = sid * rpt + ci * cpr
            pltpu.sync_copy(perm_hbm.at[pl.ds(base, cpr)], idx_v)
            pltpu.sync_copy(u_hbm.at[pl.ds(base, cpr)], rows_v)
            def grp(g, _):
                pv = idx_v[pl.ds(g * 16, 16)]
                loc = pv - hbase
                ok = (loc >= 0) & (loc < half)
                idx_v[pl.ds(g * 16, 16)] = jnp.where(ok, loc, half)
                return 0
            lax.fori_loop(0, cpr // 16, grp, 0)
            pltpu.sync_copy(rows_v, sh.at[idx_v], add=True)
            return 0
        lax.fori_loop(0, nch, chunk, 0)
        plsc.subcore_barrier()
        pltpu.sync_copy(sh.at[pl.ds(sid * sl, sl)],
                        out_hbm.at[pl.ds(hbase + sid * sl, sl)])

    return uk


_unpool1 = _make_unpool(K0, K1)
_unpool0 = _make_unpool(N0, K0)


def _fin_body(p_ref, deg_ref, b_ref, pv_ref, y_ref, s_ref, th_ref):
    deg = deg_ref[0, :] + deg_ref[1, :]
    dinv = _dinv_of(deg)
    acc = p_ref[0] + p_ref[1]
    y = jnp.maximum(acc * dinv[:, None] + b_ref[...], 0.0)
    y_ref[...] = y
    pv = pv_ref[...]
    pn = pv / jnp.sqrt(jnp.sum(pv * pv))
    s = jnp.dot(y, pn.reshape(H, 1), preferred_element_type=jnp.float32)
    s_ref[...] = s
    th_ref[...] = jnp.tanh(s)


def _fin(P, degP, b, pvec, bm=1024):
    m = P.shape[1]
    return pl.pallas_call(
        _fin_body,
        grid=(m // bm,),
        in_specs=[
            pl.BlockSpec((2, bm, H), lambda i: (0, i, 0)),
            pl.BlockSpec((2, bm), lambda i: (0, i)),
            pl.BlockSpec((1, H), lambda i: (0, 0)),
            pl.BlockSpec((1, H), lambda i: (0, 0)),
        ],
        out_specs=[
            pl.BlockSpec((bm, H), lambda i: (i, 0)),
            pl.BlockSpec((bm, 1), lambda i: (i, 0)),
            pl.BlockSpec((bm, 1), lambda i: (i, 0)),
        ],
        out_shape=[
            jax.ShapeDtypeStruct((m, H), jnp.float32),
            jax.ShapeDtypeStruct((m, 1), jnp.float32),
            jax.ShapeDtypeStruct((m, 1), jnp.float32),
        ],
    )(P, degP, b.reshape(1, H), pvec.reshape(1, H))


def _sq_body(nsteps, inject, twout, r_ref, ct_ref, pr_ref, pc_ref,
             a_ref, deg_ref, *maybe_at):
    i, j, k = pl.program_id(0), pl.program_id(1), pl.program_id(2)

    @pl.when(k == 0)
    def _():
        a_ref[...] = jnp.zeros_like(a_ref)

    rb = r_ref[...]
    cb = ct_ref[...]
    if inject:
        bk = rb.shape[1]
        colk = lax.broadcasted_iota(jnp.int32, (rb.shape[0], bk), 1) + k * bk
        rb = rb + jnp.where(colk == pr_ref[...][:, None], 1.0, 0.0)
        colk2 = lax.broadcasted_iota(jnp.int32, (cb.shape[0], bk), 1) + k * bk
        cb = cb + jnp.where(colk2 == pc_ref[...][:, None], 1.0, 0.0)
    a_ref[...] += lax.dot_general(
        rb, cb, (((1,), (1,)), ((), ())), preferred_element_type=jnp.float32)

    @pl.when(k == nsteps - 1)
    def _():
        blk = a_ref[...]
        bm, bn = blk.shape
        rows = lax.broadcasted_iota(jnp.int32, (bm, bn), 0) + i * bm
        cols = lax.broadcasted_iota(jnp.int32, (bm, bn), 1) + j * bn
        blk = jnp.where(rows == cols, 0.0, blk)
        a_ref[...] = blk
        if twout:
            maybe_at[0][...] = blk.T
        rs = jnp.sum(blk, axis=1)

        @pl.when(j == 0)
        def _():
            deg_ref[...] = rs

        @pl.when(j != 0)
        def _():
            deg_ref[...] += rs


def _square_pool(Rm, CTm, permv=None, twout=False, bm=512, bn=512, bk=2048):
    m, K = Rm.shape
    nsteps = K // bk
    grid = (m // bm, m // bn, nsteps)
    inject = permv is not None
    if permv is None:
        permv = jnp.zeros((m,), jnp.int32)
    out_specs = [
        pl.BlockSpec((bm, bn), lambda i, j, k: (i, j)),
        pl.BlockSpec((bm,), lambda i, j, k: (i,)),
    ]
    out_shape = [
        jax.ShapeDtypeStruct((m, m), jnp.float32),
        jax.ShapeDtypeStruct((m,), jnp.float32),
    ]
    if twout:
        out_specs.append(pl.BlockSpec((bn, bm), lambda i, j, k: (j, i)))
        out_shape.append(jax.ShapeDtypeStruct((m, m), jnp.float32))
    return pl.pallas_call(
        functools.partial(_sq_body, nsteps, inject, twout),
        grid=grid,
        in_specs=[
            pl.BlockSpec((bm, bk), lambda i, j, k: (i, k)),
            pl.BlockSpec((bn, bk), lambda i, j, k: (j, k)),
            pl.BlockSpec((bm,), lambda i, j, k: (i,)),
            pl.BlockSpec((bn,), lambda i, j, k: (j,)),
        ],
        out_specs=out_specs,
        out_shape=out_shape,
    )(Rm, CTm, permv, permv)


def _conv_body(a_ref, z_ref, degk_ref, degi_ref, b_ref, pv_ref,
               y_ref, s_ref, th_ref):
    dinvk = _dinv_of(degk_ref[...])
    zs = z_ref[...] * dinvk[:, None]
    acc = jnp.dot(a_ref[...], zs, preferred_element_type=jnp.float32)
    dinvi = _dinv_of(degi_ref[...])
    y = jnp.maximum(acc * dinvi[:, None] + b_ref[...], 0.0)
    y_ref[...] = y
    pv = pv_ref[...]
    pn = pv / jnp.sqrt(jnp.sum(pv * pv))
    s = jnp.dot(y, pn.reshape(H, 1), preferred_element_type=jnp.float32)
    s_ref[...] = s
    th_ref[...] = jnp.tanh(s)


def _conv_dense(A, Z, deg, b, pvec, bm=512):
    m = A.shape[0]
    return pl.pallas_call(
        _conv_body,
        grid=(m // bm,),
        in_specs=[
            pl.BlockSpec((bm, m), lambda i: (i, 0)),
            pl.BlockSpec((m, H), lambda i: (0, 0)),
            pl.BlockSpec((m,), lambda i: (0,)),
            pl.BlockSpec((bm,), lambda i: (i,)),
            pl.BlockSpec((1, H), lambda i: (0, 0)),
            pl.BlockSpec((1, H), lambda i: (0, 0)),
        ],
        out_specs=[
            pl.BlockSpec((bm, H), lambda i: (i, 0)),
            pl.BlockSpec((bm, 1), lambda i: (i, 0)),
            pl.BlockSpec((bm, 1), lambda i: (i, 0)),
        ],
        out_shape=[
            jax.ShapeDtypeStruct((m, H), jnp.float32),
            jax.ShapeDtypeStruct((m, 1), jnp.float32),
            jax.ShapeDtypeStruct((m, 1), jnp.float32),
        ],
    )(A, Z, deg, deg, b.reshape(1, H), pvec.reshape(1, H))


def _mm_body(x_ref, w_ref, si_ref, so_ref, y_ref):
    y = jnp.dot(x_ref[...] * si_ref[...][:, None], w_ref[...],
                preferred_element_type=jnp.float32)
    y_ref[...] = y * so_ref[...][:, None]


def _mm(X, W, s_in=None, s_out=None, bm=1024):
    m, f = X.shape
    if s_in is None:
        s_in = jnp.ones((m,), jnp.float32)
    if s_out is None:
        s_out = jnp.ones((m,), jnp.float32)
    return pl.pallas_call(
        _mm_body,
        grid=(m // bm,),
        in_specs=[
            pl.BlockSpec((bm, f), lambda i: (i, 0)),
            pl.BlockSpec((f, W.shape[1]), lambda i: (0, 0)),
            pl.BlockSpec((bm,), lambda i: (i,)),
            pl.BlockSpec((bm,), lambda i: (i,)),
        ],
        out_specs=pl.BlockSpec((bm, W.shape[1]), lambda i: (i, 0)),
        out_shape=jax.ShapeDtypeStruct((m, W.shape[1]), jnp.float32),
    )(X, W, s_in, s_out)


def _topk_set(score, k):
    _, perm = lax.top_k(score, k)
    return jnp.sort(perm)


def kernel(x, edge_index, edge_weight, W0, b0, W1, b1, W2, b2, U0, c0, U1, c1, p0, p1):
    xf = x.reshape(N0, H)
    dst, src = edge_index[1], edge_index[0]

    degP = _sc_deg(dst, edge_weight)
    deg0 = degP[0] + degP[1]
    dinv0 = _dinv_of(deg0)

    z0p = _mm(xf, W0, s_out=dinv0)
    P = _sc_spmm(z0p, src, dst, edge_weight)
    x0, s0, th0 = _fin(P, degP, b0, p0)
    score0 = s0.reshape(N0)

    perm0 = _topk_set(score0, K0)
    rank0 = jnp.full((N0,), -1, jnp.int32).at[perm0].set(
        jnp.arange(K0, dtype=jnp.int32))

    src2 = src.reshape(_ECH, _CH)
    dst2 = dst.reshape(_ECH, _CH)
    w2 = edge_weight.reshape(_ECH, _CH)
    rkd2 = rank0[dst].reshape(_ECH, _CH)
    rks2 = rank0[src].reshape(_ECH, _CH)
    R0f, CT0f = _sc_factors(src2, dst2, w2, rkd2, rks2, perm0)
    R0 = R0f.reshape(K0, N0)
    CT0 = CT0f.reshape(K0, N0)

    A1, deg1, A1T = _square_pool(R0, CT0, twout=True)

    x0g, t0, di0g = _gather_x0(perm0, x0, th0.reshape(N0), dinv0)
    x1, s1, th1 = _conv_dense(A1, _mm(x0g, W1, s_in=t0), deg1, b1, p1)
    score1 = s1.reshape(K0)

    perm1 = _topk_set(score1, K1)

    x1g, t1 = _gather_x1(perm1, x1, th1.reshape(K0))

    R1 = _gather_a1(perm1, A1)[0]
    CT1 = _gather_a1(perm1, A1T)[0]

    A2, deg2 = _square_pool(R1, CT1, permv=perm1, bm=512, bn=512, bk=2048)

    x2, _, _ = _conv_dense(A2, _mm(x1g, W2, s_in=t1, bm=512), deg2, b2, p1)

    z1 = _unpool1(_mm(x1, U0[:H]), _mm(x2, U0[H:]), perm1)
    x3, _, _ = _conv_dense(A1, z1, deg1, c0, p1)

    u0 = _mm(x3, U1[H:], s_out=di0g)
    zup = _unpool0(_mm(x0, U1[:H], s_out=dinv0), u0, perm0)
    Q = _sc_spmm(zup, src, dst, edge_weight)
    out, _, _ = _fin(Q, degP, c1, p1)
    return out.reshape(1, N0, H)
